# baseline (speedup 1.0000x reference)
import jax
import jax.numpy as jnp
from jax import lax
from jax.experimental import pallas as pl
from jax.experimental.pallas import tpu as pltpu
from jax.experimental.pallas import tpu_sc as plsc

B, L, V, D, C = 4096, 50, 1000000, 64, 10
LP = 56
NW = 32
SENT_W = B // NW
NB = 4
NGROUP = SENT_W // NB


def _widx(word):
    w = jnp.concatenate(
        [word, word[:, L - (LP - L):],
         jnp.zeros((B, 128 - LP), jnp.int32)], axis=1)
    return w.reshape(NW, SENT_W, 128)


def _sc_gather_body(word_hbm, table_hbm, out_hbm, idx_v, rows_v, gsem, wsem):
    cid = lax.axis_index("c")
    sid = lax.axis_index("s")
    wid = sid * 2 + cid
    base_s = wid * SENT_W
    pltpu.sync_copy(word_hbm.at[wid], idx_v)

    def do_group(g, buf):
        s0 = g * NB
        gets = [
            pltpu.async_copy(
                table_hbm.at[idx_v.at[s0 + t, pl.ds(0, LP)]],
                rows_v.at[buf, t],
                gsem,
            )
            for t in range(NB)
        ]
        for cp in gets:
            cp.wait()
        return pltpu.async_copy(
            rows_v.at[buf], out_hbm.at[pl.ds(base_s + s0, NB)], wsem
        )

    put0 = do_group(0, 0)
    put1 = do_group(1, 1)

    def pair(h, carry):
        g = 2 * h
        put0.wait()
        do_group(g + 2, 0)
        put1.wait()
        do_group(g + 3, 1)
        return carry

    lax.fori_loop(0, (NGROUP - 2) // 2, pair, 0)
    put0.wait()
    put1.wait()


def _sc_gather(widx, table):
    mesh = plsc.VectorSubcoreMesh(core_axis_name="c", subcore_axis_name="s")
    kern = pl.kernel(
        _sc_gather_body,
        mesh=mesh,
        out_type=jax.ShapeDtypeStruct((B, LP, D), jnp.float32),
        scratch_types=[
            pltpu.VMEM((SENT_W, 128), jnp.int32),
            pltpu.VMEM((2, NB, LP, D), jnp.float32),
            pltpu.SemaphoreType.DMA,
            pltpu.SemaphoreType.DMA,
        ],
        compiler_params=pltpu.CompilerParams(use_tc_tiling_on_sc=False),
    )
    return kern(widx, table)


def _tc_combine_body(x_ref, w_ref, b_ref, out_ref):
    bb = out_ref.shape[0]
    xf = x_ref[...].reshape(bb * LP, D)
    acc = jnp.broadcast_to(b_ref[...].reshape(1, 1, C), (bb, L, C))
    for i in range(5):
        zi = lax.dot_general(
            xf, w_ref[pl.ds(D * i, D), :], (((1,), (0,)), ((), ())),
            preferred_element_type=jnp.float32,
        ).reshape(bb, LP, C)
        lo = max(0, 2 - i)
        hi = min(L, L + 2 - i)
        acc = acc + jnp.pad(zi[:, lo + i - 2:hi + i - 2, :],
                            ((0, 0), (lo, L - hi), (0, 0)))
    out_ref[...] = acc


def _tc_combine(x, W, b2):
    bb = 256
    return pl.pallas_call(
        _tc_combine_body,
        grid=(B // bb,),
        in_specs=[
            pl.BlockSpec((bb, LP, D), lambda i: (i, 0, 0)),
            pl.BlockSpec((5 * D, C), lambda i: (0, 0)),
            pl.BlockSpec((1, C), lambda i: (0, 0)),
        ],
        out_specs=pl.BlockSpec((bb, L, C), lambda i: (i, 0, 0)),
        out_shape=jax.ShapeDtypeStruct((B, L, C), jnp.float32),
    )(x, W, b2)


def kernel(word, table, W, b):
    word = word.astype(jnp.int32)
    widx = _widx(word)
    x = _sc_gather(widx, table)
    return _tc_combine(x, W, b.reshape(1, C))


# speedup vs baseline: 1.0005x; 1.0005x over previous
import jax
import jax.numpy as jnp
from jax import lax
from jax.experimental import pallas as pl
from jax.experimental.pallas import tpu as pltpu
from jax.experimental.pallas import tpu_sc as plsc

B, L, V, D, C = 4096, 50, 1000000, 64, 10
LP = 56
NW = 32
SENT_W = B // NW
NB = 4
NGROUP = SENT_W // NB


def _widx(word):
    w = jnp.concatenate(
        [word, word[:, L - (LP - L):],
         jnp.zeros((B, 128 - LP), jnp.int32)], axis=1)
    return w.reshape(NW, SENT_W, 128)


def _sc_gather_body(word_hbm, table_hbm, out_hbm, idx_v, rows_v, gsem, wsem):
    cid = lax.axis_index("c")
    sid = lax.axis_index("s")
    wid = sid * 2 + cid
    base_s = wid * SENT_W
    pltpu.sync_copy(word_hbm.at[wid], idx_v)

    def do_group(g, buf):
        s0 = g * NB
        gets = [
            pltpu.async_copy(
                table_hbm.at[idx_v.at[s0 + t, pl.ds(0, LP)]],
                rows_v.at[buf, t],
                gsem,
            )
            for t in range(NB)
        ]
        for cp in gets:
            cp.wait()
        return pltpu.async_copy(
            rows_v.at[buf], out_hbm.at[pl.ds(base_s + s0, NB)], wsem
        )

    put0 = do_group(0, 0)
    put1 = do_group(1, 1)

    def pair(h, carry):
        g = 2 * h
        put0.wait()
        do_group(g + 2, 0)
        put1.wait()
        do_group(g + 3, 1)
        return carry

    lax.fori_loop(0, (NGROUP - 2) // 2, pair, 0)
    put0.wait()
    put1.wait()


def _sc_gather(widx, table):
    mesh = plsc.VectorSubcoreMesh(core_axis_name="c", subcore_axis_name="s")
    kern = pl.kernel(
        _sc_gather_body,
        mesh=mesh,
        out_type=jax.ShapeDtypeStruct((B, LP, D), jnp.float32),
        scratch_types=[
            pltpu.VMEM((SENT_W, 128), jnp.int32),
            pltpu.VMEM((2, NB, LP, D), jnp.float32),
            pltpu.SemaphoreType.DMA,
            pltpu.SemaphoreType.DMA,
        ],
        compiler_params=pltpu.CompilerParams(use_tc_tiling_on_sc=False),
    )
    return kern(widx, table)


def _tc_combine_body(x_ref, w_ref, b_ref, out_ref):
    bb = out_ref.shape[0]
    xf = x_ref[...].reshape(bb * LP, D)
    acc = jnp.broadcast_to(b_ref[...].reshape(1, 1, C), (bb, L, C))
    for i in range(5):
        zi = lax.dot_general(
            xf, w_ref[pl.ds(D * i, D), :], (((1,), (0,)), ((), ())),
            preferred_element_type=jnp.float32,
        ).reshape(bb, LP, C)
        lo = max(0, 2 - i)
        hi = min(L, L + 2 - i)
        acc = acc + jnp.pad(zi[:, lo + i - 2:hi + i - 2, :],
                            ((0, 0), (lo, L - hi), (0, 0)))
    out_ref[...] = acc


def _tc_combine(x, W, b2):
    bb = 256
    return pl.pallas_call(
        _tc_combine_body,
        grid=(B // bb,),
        in_specs=[
            pl.BlockSpec((bb, LP, D), lambda i: (i, 0, 0)),
            pl.BlockSpec((5 * D, C), lambda i: (0, 0)),
            pl.BlockSpec((1, C), lambda i: (0, 0)),
        ],
        out_specs=pl.BlockSpec((bb, L, C), lambda i: (i, 0, 0)),
        out_shape=jax.ShapeDtypeStruct((B, L, C), jnp.float32),
        compiler_params=pltpu.CompilerParams(needs_layout_passes=True),
    )(x, W, b2)


def kernel(word, table, W, b):
    word = word.astype(jnp.int32)
    widx = _widx(word)
    x = _sc_gather(widx, table)
    return _tc_combine(x, W, b.reshape(1, C))


# speedup vs baseline: 1.0017x; 1.0012x over previous
import jax
import jax.numpy as jnp
from jax import lax
from jax.experimental import pallas as pl
from jax.experimental.pallas import tpu as pltpu
from jax.experimental.pallas import tpu_sc as plsc

B, L, V, D, C = 4096, 50, 1000000, 64, 10
LP = 56
NW = 32
SENT_W = B // NW
NB = 4
NGROUP = SENT_W // NB


def _widx(word):
    w = jnp.concatenate(
        [word, word[:, L - (LP - L):],
         jnp.zeros((B, 128 - LP), jnp.int32)], axis=1)
    return w.reshape(NW, SENT_W, 128)


def _sc_gather_body(word_hbm, table_hbm, out_hbm, idx_v, rows_v, gsem, wsem):
    cid = lax.axis_index("c")
    sid = lax.axis_index("s")
    wid = sid * 2 + cid
    base_s = wid * SENT_W
    pltpu.sync_copy(word_hbm.at[wid], idx_v)

    def do_group(g, buf):
        s0 = g * NB
        gets = [
            pltpu.async_copy(
                table_hbm.at[idx_v.at[s0 + t, pl.ds(0, LP)]],
                rows_v.at[buf, t],
                gsem,
            )
            for t in range(NB)
        ]
        for cp in gets:
            cp.wait()
        return pltpu.async_copy(
            rows_v.at[buf], out_hbm.at[pl.ds(base_s + s0, NB)], wsem
        )

    put0 = do_group(0, 0)
    put1 = do_group(1, 1)

    def pair(h, carry):
        g = 2 * h
        put0.wait()
        do_group(g + 2, 0)
        put1.wait()
        do_group(g + 3, 1)
        return carry

    lax.fori_loop(0, (NGROUP - 2) // 2, pair, 0)
    put0.wait()
    put1.wait()


def _sc_gather(widx, table):
    mesh = plsc.VectorSubcoreMesh(core_axis_name="c", subcore_axis_name="s")
    kern = pl.kernel(
        _sc_gather_body,
        mesh=mesh,
        out_type=jax.ShapeDtypeStruct((B, LP, D), jnp.float32),
        scratch_types=[
            pltpu.VMEM((SENT_W, 128), jnp.int32),
            pltpu.VMEM((2, NB, LP, D), jnp.float32),
            pltpu.SemaphoreType.DMA,
            pltpu.SemaphoreType.DMA,
        ],
        compiler_params=pltpu.CompilerParams(use_tc_tiling_on_sc=False),
    )
    return kern(widx, table)


def _tc_combine_body(x_ref, w_ref, b_ref, out_ref):
    bb = out_ref.shape[0]
    xf = x_ref[...].reshape(bb * LP, D)
    acc = jnp.broadcast_to(b_ref[...].reshape(1, 1, C), (bb, L, C))
    for i in range(5):
        zi = lax.dot_general(
            xf, w_ref[pl.ds(D * i, D), :], (((1,), (0,)), ((), ())),
            preferred_element_type=jnp.float32,
        ).reshape(bb, LP, C)
        lo = max(0, 2 - i)
        hi = min(L, L + 2 - i)
        acc = acc + jnp.pad(zi[:, lo + i - 2:hi + i - 2, :],
                            ((0, 0), (lo, L - hi), (0, 0)))
    out_ref[...] = acc


def _tc_combine(x, W, b2):
    bb = 256
    return pl.pallas_call(
        _tc_combine_body,
        grid=(B // bb,),
        in_specs=[
            pl.BlockSpec((bb, LP, D), lambda i: (i, 0, 0)),
            pl.BlockSpec((5 * D, C), lambda i: (0, 0)),
            pl.BlockSpec((1, C), lambda i: (0, 0)),
        ],
        out_specs=pl.BlockSpec((bb, L, C), lambda i: (i, 0, 0)),
        out_shape=jax.ShapeDtypeStruct((B, L, C), jnp.float32),
    )(x, W, b2)


def kernel(word, table, W, b):
    word = word.astype(jnp.int32)
    widx = _widx(word)
    x = _sc_gather(widx, table)
    return _tc_combine(x, W, b.reshape(1, C))


# speedup vs baseline: 1.0129x; 1.0112x over previous
import jax
import jax.numpy as jnp
from jax import lax
from jax.experimental import pallas as pl
from jax.experimental.pallas import tpu as pltpu
from jax.experimental.pallas import tpu_sc as plsc

B, L, V, D, C = 4096, 50, 1000000, 64, 10
LP = 56
NW = 32
SENT_W = B // NW
NB = 8
NGROUP = SENT_W // NB


def _widx(word):
    w = jnp.concatenate(
        [word, word[:, L - (LP - L):],
         jnp.zeros((B, 64 - LP), jnp.int32)], axis=1)
    return w.reshape(NW, SENT_W, 64)


def _sc_gather_body(word_hbm, table_hbm, out_hbm, idx_v, rows_v, gsem, wsem):
    cid = lax.axis_index("c")
    sid = lax.axis_index("s")
    wid = sid * 2 + cid
    base_s = wid * SENT_W
    pltpu.sync_copy(word_hbm.at[wid], idx_v)

    def do_group(g, buf):
        s0 = g * NB
        gets = [
            pltpu.async_copy(
                table_hbm.at[idx_v.at[s0 + t, pl.ds(0, LP)]],
                rows_v.at[buf, t],
                gsem,
            )
            for t in range(NB)
        ]
        for cp in gets:
            cp.wait()
        return pltpu.async_copy(
            rows_v.at[buf], out_hbm.at[pl.ds(base_s + s0, NB)], wsem
        )

    put0 = do_group(0, 0)
    put1 = do_group(1, 1)

    def pair(h, carry):
        g = 2 * h
        put0.wait()
        do_group(g + 2, 0)
        put1.wait()
        do_group(g + 3, 1)
        return carry

    lax.fori_loop(0, (NGROUP - 2) // 2, pair, 0)
    put0.wait()
    put1.wait()


def _sc_gather(widx, table):
    mesh = plsc.VectorSubcoreMesh(core_axis_name="c", subcore_axis_name="s")
    kern = pl.kernel(
        _sc_gather_body,
        mesh=mesh,
        out_type=jax.ShapeDtypeStruct((B, LP, D), jnp.float32),
        scratch_types=[
            pltpu.VMEM((SENT_W, 64), jnp.int32),
            pltpu.VMEM((2, NB, LP, D), jnp.float32),
            pltpu.SemaphoreType.DMA,
            pltpu.SemaphoreType.DMA,
        ],
        compiler_params=pltpu.CompilerParams(use_tc_tiling_on_sc=False),
    )
    return kern(widx, table)


def _tc_combine_body(x_ref, w_ref, b_ref, out_ref):
    bb = out_ref.shape[0]
    xf = x_ref[...].reshape(bb * LP, D)
    acc = jnp.broadcast_to(b_ref[...].reshape(1, 1, C), (bb, L, C))
    for i in range(5):
        zi = lax.dot_general(
            xf, w_ref[pl.ds(D * i, D), :], (((1,), (0,)), ((), ())),
            preferred_element_type=jnp.float32,
        ).reshape(bb, LP, C)
        lo = max(0, 2 - i)
        hi = min(L, L + 2 - i)
        acc = acc + jnp.pad(zi[:, lo + i - 2:hi + i - 2, :],
                            ((0, 0), (lo, L - hi), (0, 0)))
    out_ref[...] = acc


def _tc_combine(x, W, b2):
    bb = 256
    return pl.pallas_call(
        _tc_combine_body,
        grid=(B // bb,),
        in_specs=[
            pl.BlockSpec((bb, LP, D), lambda i: (i, 0, 0)),
            pl.BlockSpec((5 * D, C), lambda i: (0, 0)),
            pl.BlockSpec((1, C), lambda i: (0, 0)),
        ],
        out_specs=pl.BlockSpec((bb, L, C), lambda i: (i, 0, 0)),
        out_shape=jax.ShapeDtypeStruct((B, L, C), jnp.float32),
    )(x, W, b2)


def kernel(word, table, W, b):
    word = word.astype(jnp.int32)
    widx = _widx(word)
    x = _sc_gather(widx, table)
    return _tc_combine(x, W, b.reshape(1, C))
